# baseline (speedup 1.0000x reference)
import functools
import numpy as np
import jax
import jax.numpy as jnp
from jax import lax
from jax.experimental import pallas as pl
from jax.experimental.pallas import tpu as pltpu
from jax.experimental.pallas import tpu_sc as plsc

_MODEL_DIM = 64
_MAX_POS = 512


def _positional_encoding(position, model_dim):
    pos = np.arange(position)[:, np.newaxis].astype(np.float32)
    i = np.arange(model_dim)[np.newaxis, :].astype(np.float32)
    angle_rates = 1.0 / np.power(10000, 2 * (i // 2) / np.float32(model_dim))
    angle_rads = pos * angle_rates
    angle_rads[:, 0::2] = np.sin(angle_rads[:, 0::2])
    angle_rads[:, 1::2] = np.cos(angle_rads[:, 1::2])
    return angle_rads.astype(np.float32)


_POS_ENC = _positional_encoding(_MAX_POS, _MODEL_DIM)


@functools.partial(jax.jit, static_argnums=(3, 4, 5))
def _embed(inp_t, table2, pos, batch, seq, dim):
    NC, NS = 2, 16
    NW = NC * NS
    b_per_w = batch // NW
    scale = float(np.sqrt(dim))
    two_d = 2 * dim

    mesh = plsc.VectorSubcoreMesh(core_axis_name="c", subcore_axis_name="s")

    @functools.partial(
        pl.kernel,
        mesh=mesh,
        compiler_params=pltpu.CompilerParams(needs_layout_passes=False),
        out_type=jax.ShapeDtypeStruct((seq, dim, batch), jnp.float32),
        scratch_types=[
            pltpu.VMEM((seq, b_per_w), jnp.int32),
            pltpu.VMEM((b_per_w,), jnp.int32),
            pltpu.VMEM((b_per_w,), jnp.int32),
            pltpu.VMEM((b_per_w, two_d + 8), jnp.float32),
            pltpu.VMEM((dim, b_per_w), jnp.float32),
            pltpu.VMEM((seq, dim), jnp.float32),
            pltpu.SemaphoreType.DMA,
        ],
    )
    def k(inpt_hbm, table_hbm, pos_hbm, out_hbm,
          idxblk_v, idxp_v, col0_v, rows_v, stage_v, pos_v, sem):
        wid = lax.axis_index("s") * NC + lax.axis_index("c")
        b0 = wid * b_per_w
        pltpu.sync_copy(inpt_hbm.at[:, pl.ds(b0, b_per_w)], idxblk_v)
        pltpu.sync_copy(pos_hbm, pos_v)
        iota16 = jnp.arange(16, dtype=jnp.int32)

        def chunk(t, carry):
            for j in range(b_per_w // 16):
                sl = pl.ds(j * 16, 16)
                v = idxblk_v[t, sl]
                idxp_v[sl] = jnp.right_shift(v, 1)
                col0_v[sl] = (v & 1) * dim
            pltpu.async_copy(
                table_hbm.at[idxp_v], rows_v.at[:, pl.ds(0, two_d)], sem
            ).wait()

            c0s = [col0_v[pl.ds(j * 16, 16)] for j in range(b_per_w // 16)]
            rowvs = [iota16 + (j * 16) for j in range(b_per_w // 16)]
            tsplat = jnp.full((16,), t, jnp.int32)

            def dloop(d, carry2):
                dsplat = jnp.full((16,), d, jnp.int32)
                ps = plsc.load_gather(pos_v, [tsplat, dsplat])
                for j in range(b_per_w // 16):
                    val = plsc.load_gather(rows_v, [rowvs[j], c0s[j] + d])
                    stage_v[d, pl.ds(j * 16, 16)] = val * scale + ps
                return carry2

            lax.fori_loop(0, dim, dloop, 0)

            pltpu.sync_copy(stage_v, out_hbm.at[t, :, pl.ds(b0, b_per_w)])
            return carry

        lax.fori_loop(0, seq, chunk, 0)

    return k(inp_t, table2, pos)


def kernel(inp, table, training):
    batch, seq = inp.shape
    dim = table.shape[1]
    pos = jnp.asarray(_POS_ENC[:seq])
    out_phys = _embed(
        inp.T, table.reshape(table.shape[0] // 2, 2 * dim), pos,
        batch, seq, dim,
    )
    return out_phys.transpose(2, 0, 1)


# speedup vs baseline: 2.3915x; 2.3915x over previous
import functools
import numpy as np
import jax
import jax.numpy as jnp
from jax import lax
from jax.experimental import pallas as pl
from jax.experimental.pallas import tpu as pltpu
from jax.experimental.pallas import tpu_sc as plsc

_MODEL_DIM = 64
_MAX_POS = 512


def _positional_encoding(position, model_dim):
    pos = np.arange(position)[:, np.newaxis].astype(np.float32)
    i = np.arange(model_dim)[np.newaxis, :].astype(np.float32)
    angle_rates = 1.0 / np.power(10000, 2 * (i // 2) / np.float32(model_dim))
    angle_rads = pos * angle_rates
    angle_rads[:, 0::2] = np.sin(angle_rads[:, 0::2])
    angle_rads[:, 1::2] = np.cos(angle_rads[:, 1::2])
    return angle_rads.astype(np.float32)


_POS_ENC = _positional_encoding(_MAX_POS, _MODEL_DIM)


@functools.partial(jax.jit, static_argnums=(3, 4, 5))
def _embed(idx_flat, table, pos2, batch, seq, dim):
    B = batch * seq
    NC, NS = 2, 16
    NW = NC * NS
    rows_per_w = B // NW
    seqs_per_chunk = 2
    chunk = seqs_per_chunk * seq
    n_chunks = rows_per_w // chunk
    n_groups = chunk // 16
    nvec = dim // 16
    scale = float(np.sqrt(dim))

    mesh = plsc.VectorSubcoreMesh(core_axis_name="c", subcore_axis_name="s")

    @functools.partial(
        pl.kernel,
        mesh=mesh,
        compiler_params=pltpu.CompilerParams(needs_layout_passes=False),
        out_type=jax.ShapeDtypeStruct((B, dim), jnp.float32),
        scratch_types=[
            pltpu.VMEM((rows_per_w,), jnp.int32),
            pltpu.VMEM((chunk, dim), jnp.float32),
            pltpu.VMEM((chunk, dim), jnp.float32),
            pltpu.SemaphoreType.DMA,
        ],
    )
    def k(idx_hbm, table_hbm, pos_hbm, out_hbm, idx_v, rows_v, pos_v, sem):
        wid = lax.axis_index("s") * NC + lax.axis_index("c")
        base = wid * rows_per_w
        pltpu.sync_copy(idx_hbm.at[pl.ds(base, rows_per_w)], idx_v)
        pltpu.sync_copy(pos_hbm, pos_v)

        def chunk_body(c, carry):
            off = c * chunk

            def fire_group(g, carry2):
                v = idx_v[pl.ds(off + g * 16, 16)]
                for l in range(16):
                    pltpu.make_async_copy(
                        table_hbm.at[pl.ds(v[l], 1)],
                        rows_v.at[pl.ds(g * 16 + l, 1)],
                        sem,
                    ).start()
                return carry2

            lax.fori_loop(0, n_groups, fire_group, 0)
            pltpu.make_async_copy(
                table_hbm.at[pl.ds(0, chunk)], rows_v, sem
            ).wait()

            def row_body(i, carry2):
                for j in range(nvec):
                    sl = pl.ds(j * 16, 16)
                    rows_v[i, sl] = rows_v[i, sl] * scale + pos_v[i, sl]
                return carry2

            lax.fori_loop(0, chunk, row_body, 0)
            pltpu.sync_copy(rows_v, out_hbm.at[pl.ds(base + off, chunk)])
            return carry

        lax.fori_loop(0, n_chunks, chunk_body, 0)

    return k(idx_flat, table, pos2)


def kernel(inp, table, training):
    batch, seq = inp.shape
    dim = table.shape[1]
    pos1 = _POS_ENC[:seq]
    pos2 = jnp.asarray(np.concatenate([pos1, pos1], axis=0))
    out = _embed(inp.reshape(-1), table, pos2, batch, seq, dim)
    return out.reshape(batch, seq, dim)
